# baseline (speedup 1.0000x reference)
import functools

import jax
import jax.numpy as jnp
from jax import lax
from jax.experimental import pallas as pl
from jax.experimental.pallas import tpu as pltpu
from jax.experimental.pallas import tpu_sc as plsc

NC = 2
NSUB = 16
NW = NC * NSUB
LANES = 16


def _prep_table(phi_ref, out_ref):
    l = jnp.log(phi_ref[...])
    w = 0.5 * (l[1, :] - l[0, :])
    c = 0.5 * jnp.sum(l)
    out_ref[0, :] = w
    out_ref[1, :] = jnp.zeros_like(w) + c


def _make_sc_kernel(Ns, N):
    RPW = Ns // NW
    CH = 32
    NCH = RPW // CH
    mesh = plsc.VectorSubcoreMesh(core_axis_name="c", subcore_axis_name="s")

    @functools.partial(
        pl.kernel,
        mesh=mesh,
        compiler_params=pltpu.CompilerParams(
            use_tc_tiling_on_sc=False, needs_layout_passes=False),
        out_type=jax.ShapeDtypeStruct((Ns,), jnp.float32),
        scratch_types=[
            pltpu.VMEM((CH * N,), jnp.int32),
            pltpu.VMEM((CH * N,), jnp.int32),
            pltpu.VMEM((N,), jnp.float32),
            pltpu.VMEM((N,), jnp.float32),
            pltpu.VMEM((RPW,), jnp.float32),
            pltpu.SemaphoreType.DMA,
            pltpu.SemaphoreType.DMA,
        ],
    )
    def sc_mf(x_hbm, wc_hbm, out_hbm, xb0, xb1, w_v, c_v, out_v, sem0, sem1):
        wid = lax.axis_index("s") * NC + lax.axis_index("c")
        elt0 = wid * (RPW * N)
        pltpu.sync_copy(wc_hbm.at[0], w_v)
        pltpu.sync_copy(wc_hbm.at[1], c_v)

        bufs = (xb0, xb1)
        sems = (sem0, sem1)
        c16 = c_v[pl.ds(0, LANES)]
        zeros = jnp.zeros((LANES,), jnp.float32)
        RT = 8
        iota = lax.broadcasted_iota(jnp.int32, (LANES,), 0)

        pending = [None, None]
        pending[0] = pltpu.async_copy(x_hbm.at[pl.ds(elt0, CH * N)], xb0, sem0)
        for ch in range(NCH):
            nxt = ch + 1
            if nxt < NCH:
                pending[nxt % 2] = pltpu.async_copy(
                    x_hbm.at[pl.ds(elt0 + nxt * (CH * N), CH * N)],
                    bufs[nxt % 2], sems[nxt % 2])
            pending[ch % 2].wait()
            xb = bufs[ch % 2]

            for half in range(CH // LANES):
                out16 = c16
                for t in range(LANES // RT):
                    r0 = half * LANES + t * RT

                    @plsc.parallel_loop(0, N // LANES, step=1, unroll=4,
                                        carry=(zeros,) * RT)
                    def accs(g, accs, xb=xb, r0=r0):
                        accs = list(accs)
                        base = g * LANES
                        wg = w_v[pl.ds(base, LANES)]
                        for r in range(RT):
                            xv = xb[pl.ds((r0 + r) * N + base, LANES)]
                            accs[r] = accs[r] + xv.astype(jnp.float32) * wg
                        return tuple(accs)
                    for r in range(RT):
                        s = jnp.sum(accs[r])
                        out16 = out16 + jnp.where(iota == (t * RT + r), s, 0.0)
                out_v[pl.ds(ch * CH + half * LANES, LANES)] = out16

        pltpu.sync_copy(out_v, out_hbm.at[pl.ds(wid * RPW, RPW)])

    return sc_mf


def kernel(x, phi):
    Ns, N = x.shape
    phi_t = phi.T
    wc = pl.pallas_call(
        _prep_table,
        out_shape=jax.ShapeDtypeStruct((2, N), jnp.float32),
    )(phi_t)
    return _make_sc_kernel(Ns, N)(x.reshape(Ns * N), wc)


# speedup vs baseline: 1.8230x; 1.8230x over previous
import functools

import jax
import jax.numpy as jnp
from jax import lax
from jax.experimental import pallas as pl
from jax.experimental.pallas import tpu as pltpu
from jax.experimental.pallas import tpu_sc as plsc

NC = 2
NSUB = 16
NW = NC * NSUB
LANES = 16


def _prep_table(phi_ref, out_ref):
    l = jnp.log(phi_ref[...])
    w = 0.5 * (l[1, :] - l[0, :])
    c = 0.5 * jnp.sum(l)
    out_ref[0, :] = w
    out_ref[1, :] = jnp.zeros_like(w) + c


def _make_sc_kernel(Ns, N):
    RPW = Ns // NW
    CH = 32
    NCH = RPW // CH
    mesh = plsc.VectorSubcoreMesh(core_axis_name="c", subcore_axis_name="s")

    @functools.partial(
        pl.kernel,
        mesh=mesh,
        compiler_params=pltpu.CompilerParams(
            use_tc_tiling_on_sc=True, needs_layout_passes=False),
        out_type=jax.ShapeDtypeStruct((Ns,), jnp.float32),
        scratch_types=[
            pltpu.VMEM((CH, N), jnp.int32),
            pltpu.VMEM((CH, N), jnp.int32),
            pltpu.VMEM((N,), jnp.float32),
            pltpu.VMEM((N,), jnp.float32),
            pltpu.VMEM((RPW,), jnp.float32),
            pltpu.SemaphoreType.DMA,
            pltpu.SemaphoreType.DMA,
        ],
    )
    def sc_mf(x_hbm, wc_hbm, out_hbm, xb0, xb1, w_v, c_v, out_v, sem0, sem1):
        wid = lax.axis_index("s") * NC + lax.axis_index("c")
        row0 = wid * RPW
        pltpu.sync_copy(wc_hbm.at[0], w_v)
        pltpu.sync_copy(wc_hbm.at[1], c_v)

        bufs = (xb0, xb1)
        sems = (sem0, sem1)
        c16 = c_v[pl.ds(0, LANES)]
        zeros = jnp.zeros((LANES,), jnp.float32)
        RT = 8
        iota = lax.broadcasted_iota(jnp.int32, (LANES,), 0)

        pending = [None, None]
        pending[0] = pltpu.async_copy(x_hbm.at[pl.ds(row0, CH)], xb0, sem0)
        for ch in range(NCH):
            nxt = ch + 1
            if nxt < NCH:
                pending[nxt % 2] = pltpu.async_copy(
                    x_hbm.at[pl.ds(row0 + nxt * CH, CH)],
                    bufs[nxt % 2], sems[nxt % 2])
            pending[ch % 2].wait()
            xb = bufs[ch % 2]

            for half in range(CH // LANES):
                out16 = c16
                for t in range(LANES // RT):
                    r0 = half * LANES + t * RT

                    @plsc.parallel_loop(0, N // LANES, step=1, unroll=4,
                                        carry=(zeros,) * RT)
                    def accs(g, accs, xb=xb, r0=r0):
                        accs = list(accs)
                        base = g * LANES
                        wg = w_v[pl.ds(base, LANES)]
                        for r in range(RT):
                            xv = xb[r0 + r, pl.ds(base, LANES)]
                            accs[r] = accs[r] + xv.astype(jnp.float32) * wg
                        return tuple(accs)
                    for r in range(RT):
                        s = jnp.sum(accs[r])
                        out16 = out16 + jnp.where(iota == (t * RT + r), s, 0.0)
                out_v[pl.ds(ch * CH + half * LANES, LANES)] = out16

        pltpu.sync_copy(out_v, out_hbm.at[pl.ds(wid * RPW, RPW)])

    return sc_mf


def kernel(x, phi):
    Ns, N = x.shape
    phi_t = phi.T
    wc = pl.pallas_call(
        _prep_table,
        out_shape=jax.ShapeDtypeStruct((2, N), jnp.float32),
    )(phi_t)
    return _make_sc_kernel(Ns, N)(x, wc)


# speedup vs baseline: 1.9220x; 1.0543x over previous
import functools

import jax
import jax.numpy as jnp
from jax import lax
from jax.experimental import pallas as pl
from jax.experimental.pallas import tpu as pltpu
from jax.experimental.pallas import tpu_sc as plsc

NC = 2
NSUB = 16
NW = NC * NSUB
LANES = 16


def _prep_table(phi_ref, out_ref):
    l = jnp.log(phi_ref[...])
    w = 0.5 * (l[1, :] - l[0, :])
    c = 0.5 * jnp.sum(l)
    out_ref[0, :] = w
    out_ref[1, :] = jnp.zeros_like(w) + c


def _make_sc_kernel(Ns, N):
    RPW = Ns // NW
    CH = 32
    NCH = RPW // CH
    mesh = plsc.VectorSubcoreMesh(core_axis_name="c", subcore_axis_name="s")

    @functools.partial(
        pl.kernel,
        mesh=mesh,
        compiler_params=pltpu.CompilerParams(
            use_tc_tiling_on_sc=True, needs_layout_passes=False),
        out_type=jax.ShapeDtypeStruct((Ns,), jnp.float32),
        scratch_types=[
            pltpu.VMEM((CH, N), jnp.int32),
            pltpu.VMEM((CH, N), jnp.int32),
            pltpu.VMEM((N,), jnp.float32),
            pltpu.VMEM((N,), jnp.float32),
            pltpu.VMEM((RPW,), jnp.float32),
            pltpu.SemaphoreType.DMA,
            pltpu.SemaphoreType.DMA,
        ],
    )
    def sc_mf(x_hbm, wc_hbm, out_hbm, xb0, xb1, w_v, c_v, out_v, sem0, sem1):
        wid = lax.axis_index("s") * NC + lax.axis_index("c")
        row0 = wid * RPW
        pltpu.sync_copy(wc_hbm.at[0], w_v)
        pltpu.sync_copy(wc_hbm.at[1], c_v)

        bufs = (xb0, xb1)
        sems = (sem0, sem1)
        c16 = c_v[pl.ds(0, LANES)]
        zeros = jnp.zeros((LANES,), jnp.float32)
        RT = 16
        iota = lax.broadcasted_iota(jnp.int32, (LANES,), 0)

        pending = [None, None]
        pending[0] = pltpu.async_copy(x_hbm.at[pl.ds(row0, CH)], xb0, sem0)
        for ch in range(NCH):
            nxt = ch + 1
            if nxt < NCH:
                pending[nxt % 2] = pltpu.async_copy(
                    x_hbm.at[pl.ds(row0 + nxt * CH, CH)],
                    bufs[nxt % 2], sems[nxt % 2])
            pending[ch % 2].wait()
            xb = bufs[ch % 2]

            for half in range(CH // LANES):
                out16 = c16
                for t in range(LANES // RT):
                    r0 = half * LANES + t * RT

                    @plsc.parallel_loop(0, N // LANES, step=1, unroll=2,
                                        carry=(zeros,) * RT)
                    def accs(g, accs, xb=xb, r0=r0):
                        accs = list(accs)
                        base = g * LANES
                        wg = w_v[pl.ds(base, LANES)]
                        for r in range(RT):
                            xv = xb[r0 + r, pl.ds(base, LANES)]
                            accs[r] = accs[r] + xv.astype(jnp.float32) * wg
                        return tuple(accs)
                    for r in range(RT):
                        s = jnp.sum(accs[r])
                        out16 = out16 + jnp.where(iota == (t * RT + r), s, 0.0)
                out_v[pl.ds(ch * CH + half * LANES, LANES)] = out16

        pltpu.sync_copy(out_v, out_hbm.at[pl.ds(wid * RPW, RPW)])

    return sc_mf


def kernel(x, phi):
    Ns, N = x.shape
    phi_t = phi.T
    wc = pl.pallas_call(
        _prep_table,
        out_shape=jax.ShapeDtypeStruct((2, N), jnp.float32),
    )(phi_t)
    return _make_sc_kernel(Ns, N)(x, wc)


# speedup vs baseline: 2.3383x; 1.2166x over previous
import functools

import jax
import jax.numpy as jnp
from jax import lax
from jax.experimental import pallas as pl
from jax.experimental.pallas import tpu as pltpu
from jax.experimental.pallas import tpu_sc as plsc

NC = 2
NSUB = 16
NW = NC * NSUB
LANES = 16


def _prep_table(phi_ref, out_ref):
    l = jnp.log(phi_ref[...])
    w = 0.5 * (l[1, :] - l[0, :])
    c = 0.5 * jnp.sum(l)
    out_ref[0, :] = w
    out_ref[1, :] = jnp.zeros_like(w) + c


def _tc_block(x_ref, wc_ref, out_ref):
    w = wc_ref[0, :]
    c = wc_ref[1, 0]
    xf = x_ref[...].astype(jnp.float32)
    out_ref[0, 0, :] = c + jnp.sum(xf * w[None, :], axis=1)


def _make_sc_kernel(Ns, N):
    RPW = Ns // NW
    CH = 32
    NCH = RPW // CH
    mesh = plsc.VectorSubcoreMesh(core_axis_name="c", subcore_axis_name="s")

    @functools.partial(
        pl.kernel,
        mesh=mesh,
        compiler_params=pltpu.CompilerParams(
            use_tc_tiling_on_sc=True, needs_layout_passes=False),
        out_type=jax.ShapeDtypeStruct((Ns,), jnp.float32),
        scratch_types=[
            pltpu.VMEM((CH, N), jnp.int32),
            pltpu.VMEM((CH, N), jnp.int32),
            pltpu.VMEM((N,), jnp.float32),
            pltpu.VMEM((N,), jnp.float32),
            pltpu.VMEM((RPW,), jnp.float32),
            pltpu.SemaphoreType.DMA,
            pltpu.SemaphoreType.DMA,
        ],
    )
    def sc_mf(x_hbm, wc_hbm, out_hbm, xb0, xb1, w_v, c_v, out_v, sem0, sem1):
        wid = lax.axis_index("s") * NC + lax.axis_index("c")
        row0 = wid * RPW
        pltpu.sync_copy(wc_hbm.at[0], w_v)
        pltpu.sync_copy(wc_hbm.at[1], c_v)

        bufs = (xb0, xb1)
        sems = (sem0, sem1)
        c16 = c_v[pl.ds(0, LANES)]
        zeros = jnp.zeros((LANES,), jnp.float32)
        RT = 16
        iota = lax.broadcasted_iota(jnp.int32, (LANES,), 0)

        pending = [None, None]
        pending[0] = pltpu.async_copy(x_hbm.at[pl.ds(row0, CH)], xb0, sem0)
        for ch in range(NCH):
            nxt = ch + 1
            if nxt < NCH:
                pending[nxt % 2] = pltpu.async_copy(
                    x_hbm.at[pl.ds(row0 + nxt * CH, CH)],
                    bufs[nxt % 2], sems[nxt % 2])
            pending[ch % 2].wait()
            xb = bufs[ch % 2]

            for half in range(CH // LANES):
                out16 = c16
                for t in range(LANES // RT):
                    r0 = half * LANES + t * RT

                    @plsc.parallel_loop(0, N // LANES, step=1, unroll=2,
                                        carry=(zeros,) * RT)
                    def accs(g, accs, xb=xb, r0=r0):
                        accs = list(accs)
                        base = g * LANES
                        wg = w_v[pl.ds(base, LANES)]
                        for r in range(RT):
                            xv = xb[r0 + r, pl.ds(base, LANES)]
                            accs[r] = accs[r] + xv.astype(jnp.float32) * wg
                        return tuple(accs)
                    for r in range(RT):
                        s = jnp.sum(accs[r])
                        out16 = out16 + jnp.where(iota == (t * RT + r), s, 0.0)
                out_v[pl.ds(ch * CH + half * LANES, LANES)] = out16

        pltpu.sync_copy(out_v, out_hbm.at[pl.ds(wid * RPW, RPW)])

    return sc_mf


def kernel(x, phi):
    Ns, N = x.shape
    phi_t = phi.T
    wc = pl.pallas_call(
        _prep_table,
        out_shape=jax.ShapeDtypeStruct((2, N), jnp.float32),
    )(phi_t)

    S = 9216
    out_sc = _make_sc_kernel(S, N)(x, wc)

    B = 256
    nb = (Ns - S) // B
    off = S // B
    out_tc = pl.pallas_call(
        _tc_block,
        grid=(nb,),
        in_specs=[
            pl.BlockSpec((B, N), lambda i: (i + off, 0)),
            pl.BlockSpec((2, N), lambda i: (0, 0)),
        ],
        out_specs=pl.BlockSpec((1, 1, B), lambda i: (i, 0, 0)),
        out_shape=jax.ShapeDtypeStruct((nb, 1, B), jnp.float32),
    )(x, wc)
    return jnp.concatenate([out_sc, out_tc.reshape(Ns - S)])


# speedup vs baseline: 4.0399x; 1.7277x over previous
import functools

import jax
import jax.numpy as jnp
from jax import lax
from jax.experimental import pallas as pl
from jax.experimental.pallas import tpu as pltpu
from jax.experimental.pallas import tpu_sc as plsc

NC = 2
NSUB = 16
NW = NC * NSUB
LANES = 16


def _prep_table(phi_ref, out_ref):
    l = jnp.log(phi_ref[...])
    w = 0.5 * (l[1, :] - l[0, :])
    c = 0.5 * jnp.sum(l)
    out_ref[0, :] = w
    out_ref[1, :] = jnp.zeros_like(w) + c


def _tc_block(x_ref, wc_ref, out_ref):
    w = wc_ref[0, :]
    c = wc_ref[1, 0]
    xf = x_ref[...].astype(jnp.float32)
    out_ref[0, 0, :] = c + jnp.sum(xf * w[None, :], axis=1)


def _make_sc_kernel(Ns, N):
    RPW = Ns // NW
    CH = 32
    NCH = RPW // CH
    mesh = plsc.VectorSubcoreMesh(core_axis_name="c", subcore_axis_name="s")

    @functools.partial(
        pl.kernel,
        mesh=mesh,
        compiler_params=pltpu.CompilerParams(
            use_tc_tiling_on_sc=True, needs_layout_passes=False),
        out_type=jax.ShapeDtypeStruct((Ns,), jnp.float32),
        scratch_types=[
            pltpu.VMEM((CH, N), jnp.int32),
            pltpu.VMEM((CH, N), jnp.int32),
            pltpu.VMEM((N,), jnp.float32),
            pltpu.VMEM((N,), jnp.float32),
            pltpu.VMEM((RPW,), jnp.float32),
            pltpu.SemaphoreType.DMA,
            pltpu.SemaphoreType.DMA,
        ],
    )
    def sc_mf(x_hbm, wc_hbm, out_hbm, xb0, xb1, w_v, c_v, out_v, sem0, sem1):
        wid = lax.axis_index("s") * NC + lax.axis_index("c")
        row0 = wid * RPW
        pltpu.sync_copy(wc_hbm.at[0], w_v)
        pltpu.sync_copy(wc_hbm.at[1], c_v)

        bufs = (xb0, xb1)
        sems = (sem0, sem1)
        c16 = c_v[pl.ds(0, LANES)]
        zeros = jnp.zeros((LANES,), jnp.float32)
        RT = 16
        iota = lax.broadcasted_iota(jnp.int32, (LANES,), 0)

        pending = [None, None]
        pending[0] = pltpu.async_copy(x_hbm.at[pl.ds(row0, CH)], xb0, sem0)
        for ch in range(NCH):
            nxt = ch + 1
            if nxt < NCH:
                pending[nxt % 2] = pltpu.async_copy(
                    x_hbm.at[pl.ds(row0 + nxt * CH, CH)],
                    bufs[nxt % 2], sems[nxt % 2])
            pending[ch % 2].wait()
            xb = bufs[ch % 2]

            for half in range(CH // LANES):
                out16 = c16
                for t in range(LANES // RT):
                    r0 = half * LANES + t * RT

                    @plsc.parallel_loop(0, N // LANES, step=1, unroll=2,
                                        carry=(zeros,) * RT)
                    def accs(g, accs, xb=xb, r0=r0):
                        accs = list(accs)
                        base = g * LANES
                        wg = w_v[pl.ds(base, LANES)]
                        for r in range(RT):
                            xv = xb[r0 + r, pl.ds(base, LANES)]
                            accs[r] = accs[r] + xv.astype(jnp.float32) * wg
                        return tuple(accs)
                    for r in range(RT):
                        s = jnp.sum(accs[r])
                        out16 = out16 + jnp.where(iota == (t * RT + r), s, 0.0)
                out_v[pl.ds(ch * CH + half * LANES, LANES)] = out16

        pltpu.sync_copy(out_v, out_hbm.at[pl.ds(wid * RPW, RPW)])

    return sc_mf


def kernel(x, phi):
    Ns, N = x.shape
    phi_t = phi.T
    wc = pl.pallas_call(
        _prep_table,
        out_shape=jax.ShapeDtypeStruct((2, N), jnp.float32),
    )(phi_t)

    S = 0
    out_sc = _make_sc_kernel(S, N)(x, wc) if S else jnp.zeros((0,), jnp.float32)

    B = 1024
    nb = (Ns - S) // B
    off = S // B
    out_tc = pl.pallas_call(
        _tc_block,
        grid=(nb,),
        in_specs=[
            pl.BlockSpec((B, N), lambda i: (i + off, 0)),
            pl.BlockSpec((2, N), lambda i: (0, 0)),
        ],
        out_specs=pl.BlockSpec((1, 1, B), lambda i: (i, 0, 0)),
        out_shape=jax.ShapeDtypeStruct((nb, 1, B), jnp.float32),
    )(x, wc)
    return jnp.concatenate([out_sc, out_tc.reshape(Ns - S)])


# speedup vs baseline: 4.6202x; 1.1436x over previous
import functools

import jax
import jax.numpy as jnp
from jax import lax
from jax.experimental import pallas as pl
from jax.experimental.pallas import tpu as pltpu
from jax.experimental.pallas import tpu_sc as plsc

NC = 2
NSUB = 16
NW = NC * NSUB
LANES = 16


def _prep_table(phi_ref, out_ref):
    l = jnp.log(phi_ref[...])
    w = 0.5 * (l[1, :] - l[0, :])
    c = 0.5 * jnp.sum(l)
    out_ref[0, :] = w
    out_ref[1, :] = jnp.zeros_like(w) + c


def _tc_block(x_ref, wc_ref, out_ref):
    w = wc_ref[0, :]
    c = wc_ref[1, 0]
    xf = x_ref[...].astype(jnp.float32)
    out_ref[0, 0, :] = c + jnp.sum(xf * w[None, :], axis=1)


def _make_sc_kernel(Ns, N):
    RPW = Ns // NW
    CH = 32
    NCH = RPW // CH
    mesh = plsc.VectorSubcoreMesh(core_axis_name="c", subcore_axis_name="s")

    @functools.partial(
        pl.kernel,
        mesh=mesh,
        compiler_params=pltpu.CompilerParams(
            use_tc_tiling_on_sc=True, needs_layout_passes=False),
        out_type=jax.ShapeDtypeStruct((Ns,), jnp.float32),
        scratch_types=[
            pltpu.VMEM((CH, N), jnp.int32),
            pltpu.VMEM((CH, N), jnp.int32),
            pltpu.VMEM((N,), jnp.float32),
            pltpu.VMEM((N,), jnp.float32),
            pltpu.VMEM((RPW,), jnp.float32),
            pltpu.SemaphoreType.DMA,
            pltpu.SemaphoreType.DMA,
        ],
    )
    def sc_mf(x_hbm, wc_hbm, out_hbm, xb0, xb1, w_v, c_v, out_v, sem0, sem1):
        wid = lax.axis_index("s") * NC + lax.axis_index("c")
        row0 = wid * RPW
        pltpu.sync_copy(wc_hbm.at[0], w_v)
        pltpu.sync_copy(wc_hbm.at[1], c_v)

        bufs = (xb0, xb1)
        sems = (sem0, sem1)
        c16 = c_v[pl.ds(0, LANES)]
        zeros = jnp.zeros((LANES,), jnp.float32)
        RT = 16
        iota = lax.broadcasted_iota(jnp.int32, (LANES,), 0)

        pending = [None, None]
        pending[0] = pltpu.async_copy(x_hbm.at[pl.ds(row0, CH)], xb0, sem0)
        for ch in range(NCH):
            nxt = ch + 1
            if nxt < NCH:
                pending[nxt % 2] = pltpu.async_copy(
                    x_hbm.at[pl.ds(row0 + nxt * CH, CH)],
                    bufs[nxt % 2], sems[nxt % 2])
            pending[ch % 2].wait()
            xb = bufs[ch % 2]

            for half in range(CH // LANES):
                out16 = c16
                for t in range(LANES // RT):
                    r0 = half * LANES + t * RT

                    @plsc.parallel_loop(0, N // LANES, step=1, unroll=2,
                                        carry=(zeros,) * RT)
                    def accs(g, accs, xb=xb, r0=r0):
                        accs = list(accs)
                        base = g * LANES
                        wg = w_v[pl.ds(base, LANES)]
                        for r in range(RT):
                            xv = xb[r0 + r, pl.ds(base, LANES)]
                            accs[r] = accs[r] + xv.astype(jnp.float32) * wg
                        return tuple(accs)
                    for r in range(RT):
                        s = jnp.sum(accs[r])
                        out16 = out16 + jnp.where(iota == (t * RT + r), s, 0.0)
                out_v[pl.ds(ch * CH + half * LANES, LANES)] = out16

        pltpu.sync_copy(out_v, out_hbm.at[pl.ds(wid * RPW, RPW)])

    return sc_mf


def kernel(x, phi):
    Ns, N = x.shape
    phi_t = phi.T
    wc = pl.pallas_call(
        _prep_table,
        out_shape=jax.ShapeDtypeStruct((2, N), jnp.float32),
    )(phi_t)

    S = 0
    out_sc = _make_sc_kernel(S, N)(x, wc) if S else jnp.zeros((0,), jnp.float32)

    B = 2048
    nb = (Ns - S) // B
    off = S // B
    out_tc = pl.pallas_call(
        _tc_block,
        grid=(nb,),
        in_specs=[
            pl.BlockSpec((B, N), lambda i: (i + off, 0)),
            pl.BlockSpec((2, N), lambda i: (0, 0)),
        ],
        out_specs=pl.BlockSpec((1, 1, B), lambda i: (i, 0, 0)),
        out_shape=jax.ShapeDtypeStruct((nb, 1, B), jnp.float32),
    )(x, wc)
    return jnp.concatenate([out_sc, out_tc.reshape(Ns - S)])
